# Initial kernel scaffold; baseline (speedup 1.0000x reference)
import jax
import jax.numpy as jnp
from jax.experimental import pallas as pl


def kernel(h, x, pos, edge_index_gate, edge_index_cand, fc_w, fc_b, ggc_w, w_ih, w_hh, b_ih, b_hh):
    raise NotImplementedError("write your pallas kernel here")


# speedup vs baseline: 4.0268x; 4.0268x over previous
import functools

import jax
import jax.numpy as jnp
from jax import lax
from jax.experimental import pallas as pl
from jax.experimental.pallas import tpu as pltpu
from jax.experimental.pallas import tpu_sc as plsc

N = 10000
E = 320000
D = 128
DG = 3 * D

NC = 2
NS = 16
NW = NC * NS
C = 128
EPT = 10112
EPAD = EPT * NW
NCHUNK = EPT // C
NPAD = 10240
RPT = NPAD // NS

R = 400
GRID = N // R


def _stage1_body(x_ref, h_ref, wx_ref, wh_ref, b_ref, ggc_ref, whh_ref,
                 bhh_ref, xx_ref, m_ref, gh_ref):
    xx = jnp.dot(x_ref[...], wx_ref[...], preferred_element_type=jnp.float32)
    xx += jnp.dot(h_ref[...], wh_ref[...], preferred_element_type=jnp.float32)
    xx = jnp.maximum(xx + b_ref[...], 0.0)
    xx_ref[...] = xx
    m_ref[...] = jnp.dot(xx, ggc_ref[...], preferred_element_type=jnp.float32)
    gh_ref[...] = jnp.dot(xx, whh_ref[...],
                          preferred_element_type=jnp.float32) + bhh_ref[...]


_stage1 = pl.pallas_call(
    _stage1_body,
    grid=(GRID,),
    in_specs=[
        pl.BlockSpec((R, D), lambda i: (i, 0)),
        pl.BlockSpec((R, D), lambda i: (i, 0)),
        pl.BlockSpec((D, D), lambda i: (0, 0)),
        pl.BlockSpec((D, D), lambda i: (0, 0)),
        pl.BlockSpec((1, D), lambda i: (0, 0)),
        pl.BlockSpec((D, D), lambda i: (0, 0)),
        pl.BlockSpec((D, DG), lambda i: (0, 0)),
        pl.BlockSpec((1, DG), lambda i: (0, 0)),
    ],
    out_specs=[
        pl.BlockSpec((R, D), lambda i: (i, 0)),
        pl.BlockSpec((R, D), lambda i: (i, 0)),
        pl.BlockSpec((R, DG), lambda i: (i, 0)),
    ],
    out_shape=[
        jax.ShapeDtypeStruct((N, D), jnp.float32),
        jax.ShapeDtypeStruct((N, D), jnp.float32),
        jax.ShapeDtypeStruct((N, DG), jnp.float32),
    ],
)


def _sc_body(m_hbm, src_hbm, dst_hbm, zeros_hbm, out0, out1,
             src_v, dst_v, rows_v, acc_sh, sem):
    c = lax.axis_index("c")
    s = lax.axis_index("s")
    wid = s * NC + c
    r0 = s * RPT

    pltpu.sync_copy(zeros_hbm.at[pl.ds(r0, RPT)], acc_sh.at[pl.ds(r0, RPT)])
    plsc.subcore_barrier()

    base = wid * EPT

    def chunk(j, carry):
        off = base + j * C
        pltpu.sync_copy(src_hbm.at[pl.ds(off, C)], src_v)
        pltpu.sync_copy(dst_hbm.at[pl.ds(off, C)], dst_v)
        pltpu.async_copy(m_hbm.at[src_v], rows_v, sem).wait()
        pltpu.sync_copy(rows_v, acc_sh.at[dst_v], add=True)
        return carry

    lax.fori_loop(0, NCHUNK, chunk, 0)
    plsc.subcore_barrier()

    @pl.when(c == 0)
    def _():
        pltpu.sync_copy(acc_sh.at[pl.ds(r0, RPT)], out0.at[pl.ds(r0, RPT)])

    @pl.when(c == 1)
    def _():
        pltpu.sync_copy(acc_sh.at[pl.ds(r0, RPT)], out1.at[pl.ds(r0, RPT)])


_sc_segsum = pl.kernel(
    _sc_body,
    out_type=(
        jax.ShapeDtypeStruct((NPAD, D), jnp.float32),
        jax.ShapeDtypeStruct((NPAD, D), jnp.float32),
    ),
    mesh=plsc.VectorSubcoreMesh(core_axis_name="c", subcore_axis_name="s"),
    scratch_types=[
        pltpu.VMEM((C,), jnp.int32),
        pltpu.VMEM((C,), jnp.int32),
        pltpu.VMEM((C, D), jnp.float32),
        pltpu.VMEM_SHARED((NPAD, D), jnp.float32),
        pltpu.SemaphoreType.DMA,
    ],
)


def _stage2_body(p0_ref, p1_ref, xx_ref, gh_ref, wih_ref, bih_ref, out_ref):
    agg = p0_ref[...] + p1_ref[...]
    gi = jnp.dot(agg, wih_ref[...],
                 preferred_element_type=jnp.float32) + bih_ref[...]
    gh = gh_ref[...]
    r = jax.nn.sigmoid(gi[:, :D] + gh[:, :D])
    z = jax.nn.sigmoid(gi[:, D:2 * D] + gh[:, D:2 * D])
    n = jnp.tanh(gi[:, 2 * D:] + r * gh[:, 2 * D:])
    out_ref[...] = (1.0 - z) * n + z * xx_ref[...]


_stage2 = pl.pallas_call(
    _stage2_body,
    grid=(GRID,),
    in_specs=[
        pl.BlockSpec((R, D), lambda i: (i, 0)),
        pl.BlockSpec((R, D), lambda i: (i, 0)),
        pl.BlockSpec((R, D), lambda i: (i, 0)),
        pl.BlockSpec((R, DG), lambda i: (i, 0)),
        pl.BlockSpec((D, DG), lambda i: (0, 0)),
        pl.BlockSpec((1, DG), lambda i: (0, 0)),
    ],
    out_specs=pl.BlockSpec((R, D), lambda i: (i, 0)),
    out_shape=jax.ShapeDtypeStruct((N, D), jnp.float32),
)


def kernel(h, x, pos, edge_index_gate, edge_index_cand,
           fc_w, fc_b, ggc_w, w_ih, w_hh, b_ih, b_hh):
    src = edge_index_gate[0].astype(jnp.int32)
    dst = edge_index_gate[1].astype(jnp.int32)
    pad = EPAD - E
    src_p = jnp.concatenate([src, jnp.zeros((pad,), jnp.int32)])
    dst_p = jnp.concatenate([dst, jnp.full((pad,), N, jnp.int32)])

    wx = fc_w[:, :D].T
    wh = fc_w[:, D:].T
    whh = w_hh.T
    wih = w_ih.T
    b = fc_b.reshape(1, D)
    bhh = b_hh.reshape(1, DG)
    bih = b_ih.reshape(1, DG)

    xx, m, gh = _stage1(x, h, wx, wh, b, ggc_w, whh, bhh)

    zeros = jnp.zeros((NPAD, D), jnp.float32)
    p0, p1 = _sc_segsum(m, src_p, dst_p, zeros)

    return _stage2(p0, p1, xx, gh, wih, bih)
